# baseline (speedup 1.0000x reference)
import functools

import jax
import jax.numpy as jnp
from jax import lax
from jax.experimental import pallas as pl
from jax.experimental.pallas import tpu as pltpu
from jax.experimental.pallas import tpu_sc as plsc

F = 4
B = 4096
L = 12
V = 100000
D = 64
LANES = 16

NW = 32
ROWS_PER_W = B // NW
RC = 16
NCHUNK = ROWS_PER_W // RC
CB = RC * F
IPC = CB * L
SEG = RC * L
NJ = IPC // 128


def _sc_body(tab, pw, idx, lens, pred, idx_raw, idx_adj, rows, len_v, pw_v,
             out_v, sem_a, sem_b):
    wid = lax.axis_index("c") * 16 + lax.axis_index("s")
    row_base = wid * ROWS_PER_W
    sems = (sem_a, sem_b)
    pltpu.sync_copy(pw, pw_v)

    def stage(ci, buf):
        row0 = row_base + ci * RC
        for f in range(F):
            pltpu.sync_copy(lens.at[pl.ds(f * B + row0, RC)],
                            len_v.at[buf].at[pl.ds(f * RC, RC)])
            pltpu.sync_copy(idx.at[pl.ds((f * B + row0) * L, SEG)],
                            idx_raw.at[pl.ds(f * SEG, SEG)])
        for k in range(IPC // LANES):
            idx_adj[buf, k // 8, pl.ds((k % 8) * LANES, LANES)] = (
                idx_raw[pl.ds(k * LANES, LANES)]
                + (k // (SEG // LANES)) * V)
        return [
            pltpu.async_copy(tab.at[idx_adj.at[buf, j]],
                             rows.at[buf].at[pl.ds(j * 128, 128)],
                             sems[buf])
            for j in range(NJ)
        ]

    def drain(buf):
        for j in range(NJ):
            pltpu.make_async_copy(tab.at[idx_adj.at[buf, j]],
                                  rows.at[buf].at[pl.ds(j * 128, 128)],
                                  sems[buf]).wait()

    def compute(ci, buf):
        row0 = row_base + ci * RC

        def feat_body(g, carry2):
            pwg = pw_v[pl.ds(g * LANES, LANES)]
            pw_s = [pwg[l] for l in range(L)]
            len16 = len_v[buf, pl.ds(g * RC, RC)]
            for b2 in range(RC):
                ln = len16[b2]
                base = (g * RC + b2) * L
                accs = [None] * (D // LANES)
                for l in range(L):
                    w_l = jnp.where(l < ln, pw_s[l], 0.0)
                    for c in range(D // LANES):
                        t = w_l * rows[buf, base + l, pl.ds(c * LANES, LANES)]
                        accs[c] = t if accs[c] is None else accs[c] + t
                for c in range(D // LANES):
                    out_v[b2, pl.ds(g * D + c * LANES, LANES)] = accs[c]
            return carry2

        lax.fori_loop(0, F, feat_body, 0)
        pltpu.sync_copy(out_v, pred.at[pl.ds(row0, RC)])

    stage(0, 0)
    def pair_body(i, carry):
        ci = 2 * i
        stage(ci + 1, 1)
        drain(0)
        compute(ci, 0)
        stage(ci + 2, 0)
        drain(1)
        compute(ci + 1, 1)
        return carry
    lax.fori_loop(0, NCHUNK // 2 - 1, pair_body, 0)
    stage(NCHUNK - 1, 1)
    drain(0)
    compute(NCHUNK - 2, 0)
    drain(1)
    compute(NCHUNK - 1, 1)


VT = V // 128
VTAIL = V - VT * 128
NT = F * VT


def _tr_body(tabt, tails, out, slab, outbuf, sem):
    wid = lax.axis_index("c") * 16 + lax.axis_index("s")
    row_idx = [lax.iota(jnp.int32, LANES) + 16 * c for c in range(D // LANES)]
    ntiles = (NT - wid + NW - 1) // NW

    def tile_body(jj, carry):
        t = wid + jj * NW
        f = t // VT
        v0 = pl.multiple_of((t % VT) * 128, 128)
        reads = [
            pltpu.async_copy(
                tabt.at[f, pl.ds(dg * 8, 8), pl.ds(v0, 128)],
                slab.at[pl.ds(dg * 8, 8)], sem)
            for dg in range(D // 8)
        ]
        for r in reads:
            r.wait()
        for v in range(128):
            col = jnp.full((LANES,), v, jnp.int32)
            for c in range(D // LANES):
                vals = plsc.load_gather(slab, [row_idx[c], col])
                outbuf[pl.ds(v * D + c * LANES, LANES)] = vals
        pltpu.sync_copy(outbuf,
                        out.at[pl.ds((f * V + v0) * D, 128 * D)])
        return carry

    lax.fori_loop(0, ntiles, tile_body, 0)

    @pl.when(wid >= NW - F)
    def _():
        f = wid - (NW - F)
        pltpu.sync_copy(
            tails.at[pl.ds(f * VTAIL * D, VTAIL * D)],
            outbuf.at[pl.ds(0, VTAIL * D)])
        pltpu.sync_copy(
            outbuf.at[pl.ds(0, VTAIL * D)],
            out.at[pl.ds((f * V + VT * 128) * D, VTAIL * D)])


def _sc_transpose(tables_t, tails):
    mesh = plsc.VectorSubcoreMesh(core_axis_name="c", subcore_axis_name="s")
    run = functools.partial(
        pl.kernel,
        mesh=mesh,
        compiler_params=pltpu.CompilerParams(needs_layout_passes=False),
        out_type=jax.ShapeDtypeStruct((F * V * D,), jnp.float32),
        scratch_types=[
            pltpu.VMEM((D, 128), jnp.float32),
            pltpu.VMEM((128 * D,), jnp.float32),
            pltpu.SemaphoreType.DMA,
        ],
    )(_tr_body)
    return run(tables_t, tails)


def _sc_pooled(tables_flat, pw_pad, idx_flat, lens_flat):
    mesh = plsc.VectorSubcoreMesh(core_axis_name="c", subcore_axis_name="s")
    run = functools.partial(
        pl.kernel,
        mesh=mesh,
        compiler_params=pltpu.CompilerParams(use_tc_tiling_on_sc=False),
        out_type=jax.ShapeDtypeStruct((B, F * D), jnp.float32),
        scratch_types=[
            pltpu.VMEM((IPC,), jnp.int32),
            pltpu.VMEM((2, NJ, 128), jnp.int32),
            pltpu.VMEM((2, IPC, D), jnp.float32),
            pltpu.VMEM((2, CB), jnp.int32),
            pltpu.VMEM((F * LANES,), jnp.float32),
            pltpu.VMEM((RC, F * D), jnp.float32),
            pltpu.SemaphoreType.DMA,
            pltpu.SemaphoreType.DMA,
        ],
    )(_sc_body)
    return run(tables_flat, pw_pad, idx_flat, lens_flat)


def kernel(tables, pos_weight, indices, lengths):
    tables_t = jnp.transpose(tables, (0, 2, 1))
    tails = tables[:, VT * 128:, :].reshape(F * VTAIL * D)
    tables_lin = _sc_transpose(tables_t, tails).reshape(F * V, D)
    pw_pad = jnp.zeros((F, LANES), jnp.float32).at[:, :L].set(
        pos_weight.astype(jnp.float32)).reshape(F * LANES)
    idx_flat = indices.astype(jnp.int32).reshape(F * B * L)
    lens_flat = lengths.astype(jnp.int32).reshape(F * B)
    pred = _sc_pooled(tables_lin, pw_pad, idx_flat, lens_flat)
    loss = jnp.mean(pred)
    return (loss, pred)


# speedup vs baseline: 2.6628x; 2.6628x over previous
import functools

import jax
import jax.numpy as jnp
from jax import lax
from jax.experimental import pallas as pl
from jax.experimental.pallas import tpu as pltpu
from jax.experimental.pallas import tpu_sc as plsc

F = 4
B = 4096
L = 12
V = 100000
D = 64
LANES = 16

NW = 32
ROWS_PER_W = B // NW
RC = 16
NCHUNK = ROWS_PER_W // RC
CB = RC * F
IPC = CB * L
SEG = RC * L
NJ = IPC // 128


def _sc_body(tab, pw, idx, lens, pred, idx_raw, idx_adj, rows, len_v, pw_v,
             out_v, sem_a, sem_b):
    wid = lax.axis_index("c") * 16 + lax.axis_index("s")
    row_base = wid * ROWS_PER_W
    sems = (sem_a, sem_b)
    pltpu.sync_copy(pw, pw_v)

    def stage(ci, buf):
        row0 = row_base + ci * RC
        for f in range(F):
            pltpu.sync_copy(lens.at[pl.ds(f * B + row0, RC)],
                            len_v.at[buf].at[pl.ds(f * RC, RC)])
            pltpu.sync_copy(idx.at[pl.ds((f * B + row0) * L, SEG)],
                            idx_raw.at[pl.ds(f * SEG, SEG)])
        for k in range(IPC // LANES):
            idx_adj[buf, k // 8, pl.ds((k % 8) * LANES, LANES)] = (
                idx_raw[pl.ds(k * LANES, LANES)]
                + (k // (SEG // LANES)) * V)
        for j in range(NJ):
            pltpu.async_copy(tab.at[idx_adj.at[buf, j]],
                             rows.at[buf].at[pl.ds(j * 128, 128)],
                             sems[buf])

    def drain(buf):
        for j in range(NJ):
            pltpu.make_async_copy(tab.at[idx_adj.at[buf, j]],
                                  rows.at[buf].at[pl.ds(j * 128, 128)],
                                  sems[buf]).wait()

    def compute(ci, buf):
        row0 = row_base + ci * RC

        def feat_body(g, carry2):
            pwg = pw_v[pl.ds(g * LANES, LANES)]
            pw_s = [pwg[l] for l in range(L)]
            len16 = len_v[buf, pl.ds(g * RC, RC)]
            for b2 in range(RC):
                ln = len16[b2]
                base = (g * RC + b2) * L
                accs = [None] * (D // LANES)
                for l in range(L):
                    w_l = jnp.where(l < ln, pw_s[l], 0.0)
                    for c in range(D // LANES):
                        t = w_l * rows[buf, base + l, pl.ds(c * LANES, LANES)]
                        accs[c] = t if accs[c] is None else accs[c] + t
                for c in range(D // LANES):
                    out_v[b2, pl.ds(g * D + c * LANES, LANES)] = accs[c]
            return carry2

        lax.fori_loop(0, F, feat_body, 0)
        pltpu.sync_copy(out_v, pred.at[pl.ds(row0, RC)])

    stage(0, 0)
    def pair_body(i, carry):
        ci = 2 * i
        stage(ci + 1, 1)
        drain(0)
        compute(ci, 0)
        stage(ci + 2, 0)
        drain(1)
        compute(ci + 1, 1)
        return carry
    lax.fori_loop(0, NCHUNK // 2 - 1, pair_body, 0)
    stage(NCHUNK - 1, 1)
    drain(0)
    compute(NCHUNK - 2, 0)
    drain(1)
    compute(NCHUNK - 1, 1)


def _sc_pooled(tables_flat, pw_pad, idx_flat, lens_flat):
    mesh = plsc.VectorSubcoreMesh(core_axis_name="c", subcore_axis_name="s")
    run = functools.partial(
        pl.kernel,
        mesh=mesh,
        compiler_params=pltpu.CompilerParams(use_tc_tiling_on_sc=False),
        out_type=jax.ShapeDtypeStruct((B, F * D), jnp.float32),
        scratch_types=[
            pltpu.VMEM((IPC,), jnp.int32),
            pltpu.VMEM((2, NJ, 128), jnp.int32),
            pltpu.VMEM((2, IPC, D), jnp.float32),
            pltpu.VMEM((2, CB), jnp.int32),
            pltpu.VMEM((F * LANES,), jnp.float32),
            pltpu.VMEM((RC, F * D), jnp.float32),
            pltpu.SemaphoreType.DMA,
            pltpu.SemaphoreType.DMA,
        ],
    )(_sc_body)
    return run(tables_flat, pw_pad, idx_flat, lens_flat)


def kernel(tables, pos_weight, indices, lengths):
    tables_flat = tables.reshape(F * V, D)
    pw_pad = jnp.zeros((F, LANES), jnp.float32).at[:, :L].set(
        pos_weight.astype(jnp.float32)).reshape(F * LANES)
    idx_flat = indices.astype(jnp.int32).reshape(F * B * L)
    lens_flat = lengths.astype(jnp.int32).reshape(F * B)
    pred = _sc_pooled(tables_flat, pw_pad, idx_flat, lens_flat)
    loss = jnp.mean(pred)
    return (loss, pred)
